# baseline (speedup 1.0000x reference)
import jax
import jax.numpy as jnp
from jax import lax
from jax.experimental import pallas as pl
from jax.experimental.pallas import tpu as pltpu
from jax.experimental.pallas import tpu_sc as plsc

N_DOMAIN = 26
VOCAB = 100000
D_EMB = 32
BATCH = 16384

NROWS = N_DOMAIN * D_EMB
NW = 32
LANES = 16
BCH = 4096
NBCH = BATCH // BCH


def _sc_body(
    cond_hbm, table_hbm, tail_hbm, out_hbm, row_v, cond_v, out_v, sem_in, sem_out
):
    e = lax.axis_index("s") * 2 + lax.axis_index("c")
    G = 8

    def gather_chunk(base, buf):
        for t0 in range(0, BCH // LANES, G):
            idxs = [
                cond_v[pl.ds(base + (t0 + u) * LANES, LANES)] for u in range(G)
            ]
            vals = [plsc.load_gather(row_v, [idxs[u]]) for u in range(G)]
            for u in range(G):
                out_v[buf, pl.ds((t0 + u) * LANES, LANES)] = vals[u]

    QBOUNDS = [0, 32768, 65536, 98304, 99968]

    def drain_out(r):
        pltpu.make_async_copy(
            out_v.at[0], out_hbm.at[r, pl.ds(0, BCH)], sem_out
        ).wait()
        pltpu.make_async_copy(
            out_v.at[1], out_hbm.at[r, pl.ds(0, BCH)], sem_out
        ).wait()

    def domain_body(i, carry):
        r = i * D_EMB + e
        row_cps = [
            pltpu.async_copy(
                table_hbm.at[r].at[pl.ds(QBOUNDS[q], QBOUNDS[q + 1] - QBOUNDS[q])],
                row_v.at[pl.ds(QBOUNDS[q], QBOUNDS[q + 1] - QBOUNDS[q])],
                sem_in,
            )
            for q in range(4)
        ]
        row_cps.append(
            pltpu.async_copy(
                tail_hbm.at[r], row_v.at[pl.ds(99968, 128)], sem_in
            )
        )
        cp_cond = pltpu.async_copy(cond_hbm.at[i], cond_v, sem_in)

        @pl.when(i > 0)
        def _():
            drain_out(r)

        for cp in row_cps:
            cp.wait()
        cp_cond.wait()
        for c in range(NBCH):
            buf = c % 2
            if c >= 2:
                pltpu.make_async_copy(
                    out_v.at[buf], out_hbm.at[r, pl.ds(0, BCH)], sem_out
                ).wait()
            gather_chunk(c * BCH, buf)
            pltpu.async_copy(
                out_v.at[buf], out_hbm.at[r, pl.ds(c * BCH, BCH)], sem_out
            )
        return carry

    lax.fori_loop(0, N_DOMAIN, domain_body, 0)
    drain_out(0)


@jax.jit
def _sc_gather(cond_t, table_t, tail_t):
    mesh = plsc.VectorSubcoreMesh(core_axis_name="c", subcore_axis_name="s")
    return pl.kernel(
        _sc_body,
        out_type=jax.ShapeDtypeStruct((NROWS, BATCH), jnp.float32),
        mesh=mesh,
        scratch_types=[
            pltpu.VMEM((100096,), jnp.float32),
            pltpu.VMEM((BATCH,), jnp.int32),
            pltpu.VMEM((2, BCH), jnp.float32),
            pltpu.SemaphoreType.DMA,
            pltpu.SemaphoreType.DMA,
        ],
        compiler_params=pltpu.CompilerParams(
            use_tc_tiling_on_sc=True, needs_layout_passes=False
        ),
    )(cond_t, table_t, tail_t)


def kernel(conditions, tables):
    cond_t = conditions.T
    table_t = tables.transpose(0, 2, 1).reshape(NROWS, VOCAB)
    tail_t = jnp.pad(table_t[:, 99968:], ((0, 0), (0, 96)))
    out = _sc_gather(cond_t, table_t, tail_t)
    return out.reshape(N_DOMAIN, D_EMB, BATCH).transpose(2, 0, 1)
